# baseline (speedup 1.0000x reference)
import functools

import jax
import jax.numpy as jnp
from jax import lax
from jax.experimental import pallas as pl
from jax.experimental.pallas import tpu as pltpu
from jax.experimental.pallas import tpu_sc as plsc

N = 25200
E = 504000
N_PAD = 25600
E_PAD = 524288
EPT = E_PAD // 32
NCH = EPT // 128
CH = 2048
NPAIR = E_PAD // CH // 2
NB = 1280
GN = N_PAD // NB

_SC_PARAMS = pltpu.CompilerParams(needs_layout_passes=False,
                                  use_tc_tiling_on_sc=False)


@functools.cache
def _mesh():
    return plsc.VectorSubcoreMesh(
        core_axis_name="c", subcore_axis_name="s", num_cores=2, num_subcores=16)


def _wid():
    return lax.axis_index("c") * 16 + lax.axis_index("s")


def _mlp_kernel(ew_ref, w1_ref, w2_ref, out_ref):
    t = ew_ref[...].reshape(1, -1)
    t = t @ w1_ref[...].T
    t = jnp.where(t > 0, t, jnp.exp(t) - 1.0)
    t = t @ w2_ref[...].T
    t = jnp.tanh(t)
    t = jnp.maximum(t, 0.0)
    out_ref[...] = t.reshape(-1, 1)


def _edge_mlp(edge_weight, adj_w1, adj_w2):
    return pl.pallas_call(
        _mlp_kernel,
        out_shape=jax.ShapeDtypeStruct((420, 1), jnp.float32),
    )(edge_weight, adj_w1, adj_w2)


def _dis_kernel(degs_ref, out_ref):
    deg = jnp.sum(degs_ref[...], axis=0)
    out_ref[...] = jnp.where(deg > 0, lax.rsqrt(deg), 0.0)


def _dis(deg_parts):
    return pl.pallas_call(
        _dis_kernel,
        out_shape=jax.ShapeDtypeStruct((200, 128), jnp.float32),
    )(deg_parts)


def _xpose_kernel(x_ref, out_ref):
    xt = x_ref[...].T
    z = jnp.zeros((64, N_PAD - N), jnp.float32)
    out_ref[0] = jnp.concatenate([xt[:64], z], axis=1)
    out_ref[1] = jnp.concatenate([xt[64:], z], axis=1)


def _xpose(x):
    return pl.pallas_call(
        _xpose_kernel,
        out_shape=jax.ShapeDtypeStruct((2, 64, N_PAD), jnp.float32),
    )(x)


def _unpose_kernel(a_ref, b_ref, out_ref):
    out_ref[...] = a_ref[0][:, :N].T + b_ref[...].reshape(1, 64)


def _unpose(a, bias):
    return pl.pallas_call(
        _unpose_kernel,
        out_shape=jax.ShapeDtypeStruct((N, 64), jnp.float32),
    )(a, bias.reshape(64, 1))


def _comb_kernel(*refs, has_y, has_sub):
    i = 0
    a = refs[i][...]; i += 1
    if has_y:
        a = a + refs[i][...]; i += 1
    if has_sub:
        a = a - refs[i][...]; i += 1
    refs[i][...] = a


def _comb(a, y=None, ycol=None, sub=None):
    has_y, has_sub = y is not None, sub is not None
    H = a.shape[0]
    in_specs = [pl.BlockSpec((H, 64, NB), lambda i: (0, 0, i))]
    args = [a]
    if has_y:
        in_specs.append(pl.BlockSpec((1, 64, NB), lambda i, c=ycol: (c, 0, i)))
        args.append(y)
    if has_sub:
        in_specs.append(pl.BlockSpec((H, 64, NB), lambda i: (0, 0, i)))
        args.append(sub)
    return pl.pallas_call(
        functools.partial(_comb_kernel, has_y=has_y, has_sub=has_sub),
        grid=(GN,),
        in_specs=in_specs,
        out_specs=pl.BlockSpec((H, 64, NB), lambda i: (0, 0, i)),
        out_shape=jax.ShapeDtypeStruct((H, 64, N_PAD), jnp.float32),
    )(*args)


def _conv1_mm_kernel(*refs):
    ts = refs[:5]
    w, b, out_ref = refs[5], refs[6], refs[7]
    wv = w[...]
    acc = None
    for k in range(5):
        tk = ts[k][...]
        for h in range(2):
            wk = wv[k, 64 * h:64 * h + 64, :]
            d = lax.dot_general(wk, tk[h],
                                (((0,), (0,)), ((), ())),
                                preferred_element_type=jnp.float32)
            acc = d if acc is None else acc + d
    out_ref[...] = jnp.maximum(acc + b[...], 0.0)


def _conv1_mm(ts, w, b):
    t_spec = pl.BlockSpec((2, 64, NB), lambda i: (0, 0, i))
    return pl.pallas_call(
        _conv1_mm_kernel,
        grid=(GN,),
        in_specs=[t_spec] * 5 + [
            pl.BlockSpec((5, 128, 256), lambda i: (0, 0, 0)),
            pl.BlockSpec((256, 1), lambda i: (0, 0)),
        ],
        out_specs=pl.BlockSpec((256, NB), lambda i: (0, i)),
        out_shape=jax.ShapeDtypeStruct((256, N_PAD), jnp.float32),
    )(*ts, w, b.reshape(256, 1))


def _proj_mm_kernel(h, w, out_ref):
    out_ref[0] = lax.dot_general(w[0], h[...], (((0,), (0,)), ((), ())),
                                 preferred_element_type=jnp.float32)


def _proj_mm(h, w):
    return pl.pallas_call(
        _proj_mm_kernel,
        grid=(5, GN),
        in_specs=[
            pl.BlockSpec((256, NB), lambda k, i: (0, i)),
            pl.BlockSpec((1, 256, 64), lambda k, i: (k, 0, 0)),
        ],
        out_specs=pl.BlockSpec((1, 64, NB), lambda k, i: (k, 0, i)),
        out_shape=jax.ShapeDtypeStruct((5, 64, N_PAD), jnp.float32),
    )(h, w)


def _deg_body(rowp, coefp, out, dpriv, idxb, cb):
    w = _wid()
    def zero(i, _):
        dpriv[pl.ds(16 * i, 16)] = jnp.zeros((16,), jnp.float32)
        return 0
    lax.fori_loop(0, N_PAD // 16, zero, 0)
    base = w * EPT
    def chunk(k, _):
        e0 = base + k * 128
        pltpu.sync_copy(rowp.at[pl.ds(e0, 128)], idxb)
        pltpu.sync_copy(coefp.at[pl.ds(e0, 128)], cb)
        @plsc.parallel_loop(0, 128, 16, unroll=4)
        def _(g):
            r16 = idxb[pl.ds(g, 16)]
            c16 = cb[pl.ds(g, 16)]
            plsc.addupdate_scatter(dpriv, [r16], c16)
        return 0
    lax.fori_loop(0, NCH, chunk, 0)
    pltpu.sync_copy(dpriv, out.at[pl.ds(w * N_PAD, N_PAD)])


@functools.cache
def _deg_call():
    return pl.kernel(
        _deg_body,
        out_type=jax.ShapeDtypeStruct((32 * N_PAD,), jnp.float32),
        mesh=_mesh(),
        compiler_params=_SC_PARAMS,
        scratch_types=[
            pltpu.VMEM((N_PAD,), jnp.float32),
            pltpu.VMEM((128,), jnp.int32),
            pltpu.VMEM((128,), jnp.float32),
        ],
    )


def _norm_body(dis, rowp, colp, coefp, pc1_out, pc2_out,
               disv, idxr, idxc, cb, pkb, na, nb):
    w = _wid()
    pltpu.sync_copy(dis, disv)
    base = w * EPT
    def chunk(k, _):
        e0 = base + k * 128
        pltpu.sync_copy(rowp.at[pl.ds(e0, 128)], idxr)
        pltpu.sync_copy(colp.at[pl.ds(e0, 128)], idxc)
        pltpu.sync_copy(coefp.at[pl.ds(e0, 128)], cb)
        @plsc.parallel_loop(0, 128, 16, unroll=4)
        def _(g):
            sl = pl.ds(g, 16)
            r16 = idxr[sl]
            c16 = idxc[sl]
            w16 = cb[sl]
            dr = plsc.load_gather(disv, [r16])
            dc = plsc.load_gather(disv, [c16])
            v = -(dr * w16 * dc)
            pkb[sl] = jnp.bitwise_or(r16, jnp.left_shift(c16, 16))
            na[sl] = plsc.bitcast(v, jnp.int32)
            nb[sl] = plsc.bitcast(v + v, jnp.int32)
        pltpu.sync_copy(pkb, pc1_out.at[0, pl.ds(e0, 128)])
        pltpu.sync_copy(pkb, pc2_out.at[0, pl.ds(e0, 128)])
        pltpu.sync_copy(na, pc1_out.at[1, pl.ds(e0, 128)])
        pltpu.sync_copy(nb, pc2_out.at[1, pl.ds(e0, 128)])
        return 0
    lax.fori_loop(0, NCH, chunk, 0)


@functools.cache
def _norm_call():
    return pl.kernel(
        _norm_body,
        out_type=(jax.ShapeDtypeStruct((2, E_PAD), jnp.int32),
                  jax.ShapeDtypeStruct((2, E_PAD), jnp.int32)),
        mesh=_mesh(),
        compiler_params=_SC_PARAMS,
        scratch_types=[
            pltpu.VMEM((N_PAD,), jnp.float32),
            pltpu.VMEM((128,), jnp.int32),
            pltpu.VMEM((128,), jnp.int32),
            pltpu.VMEM((128,), jnp.float32),
            pltpu.VMEM((128,), jnp.int32),
            pltpu.VMEM((128,), jnp.int32),
            pltpu.VMEM((128,), jnp.int32),
        ],
    )


def _lxt_body(vT, pcp, out, vbuf, abuf, pc0, pc1, s0, s1, *, nh):
    w = _wid()
    f0 = jnp.zeros((16,), jnp.int32)
    f1 = jnp.full((16,), 1, jnp.int32)

    def issue(k, pcb, sem):
        pltpu.async_copy(pcp.at[:, pl.ds(k * CH, CH)], pcb, sem)

    def process(k, pcb, sem, more):
        pltpu.make_async_copy(pcp.at[:, pl.ds(k * CH, CH)], pcb, sem).wait()
        @plsc.parallel_loop(0, CH, 64, unroll=4)
        def _(i):
            for u in range(4):
                sl = pl.ds(i + 16 * u, 16)
                pk16 = pcb[0, sl]
                c16 = plsc.bitcast(pcb[1, sl], jnp.float32)
                r16 = jnp.bitwise_and(pk16, 0xFFFF)
                o16 = lax.shift_right_logical(pk16, 16)
                v0 = plsc.load_gather(vbuf, [f0, r16])
                plsc.addupdate_scatter(abuf, [f0, o16], v0 * c16)
                v1 = plsc.load_gather(vbuf, [f1, r16])
                plsc.addupdate_scatter(abuf, [f1, o16], v1 * c16)
        @pl.when(more)
        def _():
            issue(k + 2, pcb, sem)

    for h in range(nh):
        pltpu.sync_copy(vT.at[h, w], vbuf)
        def zero(i, _):
            sl = pl.ds(16 * i, 16)
            z = jnp.zeros((16,), jnp.float32)
            abuf[0, sl] = z
            abuf[1, sl] = z
            return 0
        lax.fori_loop(0, N_PAD // 16, zero, 0)
        issue(0, pc0, s0)
        issue(1, pc1, s1)
        def pair(m, _):
            more = m < NPAIR - 1
            process(2 * m, pc0, s0, more)
            process(2 * m + 1, pc1, s1, more)
            return 0
        lax.fori_loop(0, NPAIR, pair, 0)
        pltpu.sync_copy(abuf, out.at[h, w])


@functools.cache
def _lxt_call(nh):
    return pl.kernel(
        functools.partial(_lxt_body, nh=nh),
        out_type=jax.ShapeDtypeStruct((nh, 32, 2, N_PAD), jnp.float32),
        mesh=_mesh(),
        compiler_params=_SC_PARAMS,
        scratch_types=[
            pltpu.VMEM((2, N_PAD), jnp.float32),
            pltpu.VMEM((2, N_PAD), jnp.float32),
            pltpu.VMEM((2, CH), jnp.int32),
            pltpu.VMEM((2, CH), jnp.int32),
            pltpu.SemaphoreType.DMA,
            pltpu.SemaphoreType.DMA,
        ],
    )


def _lx(vT, pc):
    nh = vT.shape[0]
    v32 = vT.reshape(nh, 32, 2, N_PAD)
    return _lxt_call(nh)(v32, pc).reshape(nh, 64, N_PAD)


def kernel(x, edge_index, edge_weight, adj_w1, adj_w2, conv1_w, conv1_b, conv2_w, conv2_b):
    ew = _edge_mlp(edge_weight, adj_w1, adj_w2)
    reps = edge_index.shape[-1] // 420
    train_ew = jnp.tile(ew, (reps, 1))

    pad = E_PAD - E
    rowp = jnp.concatenate([edge_index[0], jnp.zeros((pad,), edge_index.dtype)])
    colp = jnp.concatenate([edge_index[1], jnp.zeros((pad,), edge_index.dtype)])
    cp = jnp.concatenate([train_ew.reshape(-1), jnp.zeros((pad,), jnp.float32)])

    deg_parts = _deg_call()(rowp, cp).reshape(32, 200, 128)
    dis = _dis(deg_parts).reshape(N_PAD)
    pc1, pc2 = _norm_call()(dis, rowp, colp, cp)

    t0 = _xpose(x)
    t1 = _lx(t0, pc1)
    t2 = _comb(_lx(t1, pc2), sub=t0)
    t3 = _comb(_lx(t2, pc2), sub=t1)
    t4 = _comb(_lx(t3, pc2), sub=t2)
    h = _conv1_mm([t0, t1, t2, t3, t4], conv1_w, conv1_b)

    yy = _proj_mm(h, conv2_w)
    b4 = yy[4:5]
    b3 = _comb(_lx(b4, pc2), y=yy, ycol=3)
    b2 = _comb(_lx(b3, pc2), y=yy, ycol=2, sub=b4)
    b1 = _comb(_lx(b2, pc2), y=yy, ycol=1, sub=b3)
    outT = _comb(_lx(b1, pc1), y=yy, ycol=0, sub=b2)
    out = _unpose(outT, conv2_b)

    return (out, ew, train_ew)


# speedup vs baseline: 1.0323x; 1.0323x over previous
import functools

import jax
import jax.numpy as jnp
from jax import lax
from jax.experimental import pallas as pl
from jax.experimental.pallas import tpu as pltpu
from jax.experimental.pallas import tpu_sc as plsc

N = 25200
E = 504000
N_PAD = 25600
E_PAD = 524288
EPT = E_PAD // 32
NCH = EPT // 128
CH = 2048
NPAIR = E_PAD // CH // 2
NB = 1280
GN = N_PAD // NB

_SC_PARAMS = pltpu.CompilerParams(needs_layout_passes=False,
                                  use_tc_tiling_on_sc=False)


@functools.cache
def _mesh():
    return plsc.VectorSubcoreMesh(
        core_axis_name="c", subcore_axis_name="s", num_cores=2, num_subcores=16)


def _wid():
    return lax.axis_index("c") * 16 + lax.axis_index("s")


def _mlp_kernel(ew_ref, w1_ref, w2_ref, out_ref):
    t = ew_ref[...].reshape(1, -1)
    t = t @ w1_ref[...].T
    t = jnp.where(t > 0, t, jnp.exp(t) - 1.0)
    t = t @ w2_ref[...].T
    t = jnp.tanh(t)
    t = jnp.maximum(t, 0.0)
    out_ref[...] = t.reshape(-1, 1)


def _edge_mlp(edge_weight, adj_w1, adj_w2):
    return pl.pallas_call(
        _mlp_kernel,
        out_shape=jax.ShapeDtypeStruct((420, 1), jnp.float32),
    )(edge_weight, adj_w1, adj_w2)


def _dis_kernel(degs_ref, out_ref):
    deg = jnp.sum(degs_ref[...], axis=0)
    out_ref[...] = jnp.where(deg > 0, lax.rsqrt(deg), 0.0)


def _dis(deg_parts):
    return pl.pallas_call(
        _dis_kernel,
        out_shape=jax.ShapeDtypeStruct((200, 128), jnp.float32),
    )(deg_parts)


def _xpose_kernel(x_ref, out_ref):
    xt = x_ref[...].T
    z = jnp.zeros((64, N_PAD - N), jnp.float32)
    out_ref[0] = jnp.concatenate([xt[:64], z], axis=1)
    out_ref[1] = jnp.concatenate([xt[64:], z], axis=1)


def _xpose(x):
    return pl.pallas_call(
        _xpose_kernel,
        out_shape=jax.ShapeDtypeStruct((2, 64, N_PAD), jnp.float32),
    )(x)


def _unpose_kernel(a_ref, b_ref, out_ref):
    out_ref[...] = a_ref[0][:, :N].T + b_ref[...].reshape(1, 64)


def _unpose(a, bias):
    return pl.pallas_call(
        _unpose_kernel,
        out_shape=jax.ShapeDtypeStruct((N, 64), jnp.float32),
    )(a, bias.reshape(64, 1))


def _comb_kernel(*refs, has_y, has_sub):
    i = 0
    a = refs[i][...]; i += 1
    if has_y:
        a = a + refs[i][...]; i += 1
    if has_sub:
        a = a - refs[i][...]; i += 1
    refs[i][...] = a


def _comb(a, y=None, ycol=None, sub=None):
    has_y, has_sub = y is not None, sub is not None
    H = a.shape[0]
    in_specs = [pl.BlockSpec((H, 64, NB), lambda i: (0, 0, i))]
    args = [a]
    if has_y:
        in_specs.append(pl.BlockSpec((1, 64, NB), lambda i, c=ycol: (c, 0, i)))
        args.append(y)
    if has_sub:
        in_specs.append(pl.BlockSpec((H, 64, NB), lambda i: (0, 0, i)))
        args.append(sub)
    return pl.pallas_call(
        functools.partial(_comb_kernel, has_y=has_y, has_sub=has_sub),
        grid=(GN,),
        in_specs=in_specs,
        out_specs=pl.BlockSpec((H, 64, NB), lambda i: (0, 0, i)),
        out_shape=jax.ShapeDtypeStruct((H, 64, N_PAD), jnp.float32),
    )(*args)


def _conv1_mm_kernel(*refs):
    ts = refs[:5]
    w, b, out_ref = refs[5], refs[6], refs[7]
    wv = w[...]
    acc = None
    for k in range(5):
        tk = ts[k][...]
        for h in range(2):
            wk = wv[k, 64 * h:64 * h + 64, :]
            d = lax.dot_general(wk, tk[h],
                                (((0,), (0,)), ((), ())),
                                preferred_element_type=jnp.float32)
            acc = d if acc is None else acc + d
    out_ref[...] = jnp.maximum(acc + b[...], 0.0)


def _conv1_mm(ts, w, b):
    t_spec = pl.BlockSpec((2, 64, NB), lambda i: (0, 0, i))
    return pl.pallas_call(
        _conv1_mm_kernel,
        grid=(GN,),
        in_specs=[t_spec] * 5 + [
            pl.BlockSpec((5, 128, 256), lambda i: (0, 0, 0)),
            pl.BlockSpec((256, 1), lambda i: (0, 0)),
        ],
        out_specs=pl.BlockSpec((256, NB), lambda i: (0, i)),
        out_shape=jax.ShapeDtypeStruct((256, N_PAD), jnp.float32),
    )(*ts, w, b.reshape(256, 1))


def _proj_mm_kernel(h, w, out_ref):
    out_ref[0] = lax.dot_general(w[0], h[...], (((0,), (0,)), ((), ())),
                                 preferred_element_type=jnp.float32)


def _proj_mm(h, w):
    return pl.pallas_call(
        _proj_mm_kernel,
        grid=(5, GN),
        in_specs=[
            pl.BlockSpec((256, NB), lambda k, i: (0, i)),
            pl.BlockSpec((1, 256, 64), lambda k, i: (k, 0, 0)),
        ],
        out_specs=pl.BlockSpec((1, 64, NB), lambda k, i: (k, 0, i)),
        out_shape=jax.ShapeDtypeStruct((5, 64, N_PAD), jnp.float32),
    )(h, w)


def _deg_body(rowp, coefp, out, dpriv, idxb, cb):
    w = _wid()
    def zero(i, _):
        dpriv[pl.ds(16 * i, 16)] = jnp.zeros((16,), jnp.float32)
        return 0
    lax.fori_loop(0, N_PAD // 16, zero, 0)
    base = w * EPT
    def chunk(k, _):
        e0 = base + k * 128
        pltpu.sync_copy(rowp.at[pl.ds(e0, 128)], idxb)
        pltpu.sync_copy(coefp.at[pl.ds(e0, 128)], cb)
        @plsc.parallel_loop(0, 128, 16, unroll=4)
        def _(g):
            r16 = idxb[pl.ds(g, 16)]
            c16 = cb[pl.ds(g, 16)]
            plsc.addupdate_scatter(dpriv, [r16], c16)
        return 0
    lax.fori_loop(0, NCH, chunk, 0)
    pltpu.sync_copy(dpriv, out.at[pl.ds(w * N_PAD, N_PAD)])


@functools.cache
def _deg_call():
    return pl.kernel(
        _deg_body,
        out_type=jax.ShapeDtypeStruct((32 * N_PAD,), jnp.float32),
        mesh=_mesh(),
        compiler_params=_SC_PARAMS,
        scratch_types=[
            pltpu.VMEM((N_PAD,), jnp.float32),
            pltpu.VMEM((128,), jnp.int32),
            pltpu.VMEM((128,), jnp.float32),
        ],
    )


def _norm_body(dis, rowp, colp, coefp, pc1_out, pc2_out,
               disv, idxr, idxc, cb, pkb, na, nb):
    w = _wid()
    pltpu.sync_copy(dis, disv)
    base = w * EPT
    def chunk(k, _):
        e0 = base + k * 128
        pltpu.sync_copy(rowp.at[pl.ds(e0, 128)], idxr)
        pltpu.sync_copy(colp.at[pl.ds(e0, 128)], idxc)
        pltpu.sync_copy(coefp.at[pl.ds(e0, 128)], cb)
        @plsc.parallel_loop(0, 128, 16, unroll=4)
        def _(g):
            sl = pl.ds(g, 16)
            r16 = idxr[sl]
            c16 = idxc[sl]
            w16 = cb[sl]
            dr = plsc.load_gather(disv, [r16])
            dc = plsc.load_gather(disv, [c16])
            v = -(dr * w16 * dc)
            pkb[sl] = jnp.bitwise_or(r16, jnp.left_shift(c16, 16))
            na[sl] = plsc.bitcast(v, jnp.int32)
            nb[sl] = plsc.bitcast(v + v, jnp.int32)
        pltpu.sync_copy(pkb, pc1_out.at[0, pl.ds(e0, 128)])
        pltpu.sync_copy(pkb, pc2_out.at[0, pl.ds(e0, 128)])
        pltpu.sync_copy(na, pc1_out.at[1, pl.ds(e0, 128)])
        pltpu.sync_copy(nb, pc2_out.at[1, pl.ds(e0, 128)])
        return 0
    lax.fori_loop(0, NCH, chunk, 0)


@functools.cache
def _norm_call():
    return pl.kernel(
        _norm_body,
        out_type=(jax.ShapeDtypeStruct((2, E_PAD), jnp.int32),
                  jax.ShapeDtypeStruct((2, E_PAD), jnp.int32)),
        mesh=_mesh(),
        compiler_params=_SC_PARAMS,
        scratch_types=[
            pltpu.VMEM((N_PAD,), jnp.float32),
            pltpu.VMEM((128,), jnp.int32),
            pltpu.VMEM((128,), jnp.int32),
            pltpu.VMEM((128,), jnp.float32),
            pltpu.VMEM((128,), jnp.int32),
            pltpu.VMEM((128,), jnp.int32),
            pltpu.VMEM((128,), jnp.int32),
        ],
    )


def _lxt_body(*refs, nh, has_y, has_sub):
    i = 0
    vT = refs[i]; i += 1
    pcp = refs[i]; i += 1
    y = sub = None
    if has_y:
        y = refs[i]; i += 1
    if has_sub:
        sub = refs[i]; i += 1
    out, vbuf, abuf, pc0, pc1, s0, s1 = refs[i:i + 7]
    w = _wid()
    f0 = jnp.zeros((16,), jnp.int32)
    f1 = jnp.full((16,), 1, jnp.int32)

    def issue(k, pcb, sem):
        pltpu.async_copy(pcp.at[:, pl.ds(k * CH, CH)], pcb, sem)

    def process(k, pcb, sem, more):
        pltpu.make_async_copy(pcp.at[:, pl.ds(k * CH, CH)], pcb, sem).wait()
        @plsc.parallel_loop(0, CH, 64, unroll=4)
        def _(i):
            for u in range(4):
                sl = pl.ds(i + 16 * u, 16)
                pk16 = pcb[0, sl]
                c16 = plsc.bitcast(pcb[1, sl], jnp.float32)
                r16 = jnp.bitwise_and(pk16, 0xFFFF)
                o16 = lax.shift_right_logical(pk16, 16)
                v0 = plsc.load_gather(vbuf, [f0, r16])
                plsc.addupdate_scatter(abuf, [f0, o16], v0 * c16)
                v1 = plsc.load_gather(vbuf, [f1, r16])
                plsc.addupdate_scatter(abuf, [f1, o16], v1 * c16)
        @pl.when(more)
        def _():
            issue(k + 2, pcb, sem)

    for h in range(nh):
        pltpu.sync_copy(vT.at[h, w], vbuf)
        def zero(i, _):
            sl = pl.ds(16 * i, 16)
            z = jnp.zeros((16,), jnp.float32)
            abuf[0, sl] = z
            abuf[1, sl] = z
            return 0
        lax.fori_loop(0, N_PAD // 16, zero, 0)
        issue(0, pc0, s0)
        issue(1, pc1, s1)
        def pair(m, _):
            more = m < NPAIR - 1
            process(2 * m, pc0, s0, more)
            process(2 * m + 1, pc1, s1, more)
            return 0
        lax.fori_loop(0, NPAIR, pair, 0)
        for arr, sign in ((y, 1.0), (sub, -1.0)):
            if arr is not None:
                pltpu.sync_copy(arr.at[h, w], vbuf)
                @plsc.parallel_loop(0, N_PAD, 16, unroll=4)
                def _(i):
                    sl = pl.ds(i, 16)
                    if sign > 0:
                        abuf[0, sl] = abuf[0, sl] + vbuf[0, sl]
                        abuf[1, sl] = abuf[1, sl] + vbuf[1, sl]
                    else:
                        abuf[0, sl] = abuf[0, sl] - vbuf[0, sl]
                        abuf[1, sl] = abuf[1, sl] - vbuf[1, sl]
        pltpu.sync_copy(abuf, out.at[h, w])


@functools.cache
def _lxt_call(nh, has_y, has_sub):
    return pl.kernel(
        functools.partial(_lxt_body, nh=nh, has_y=has_y, has_sub=has_sub),
        out_type=jax.ShapeDtypeStruct((nh, 32, 2, N_PAD), jnp.float32),
        mesh=_mesh(),
        compiler_params=_SC_PARAMS,
        scratch_types=[
            pltpu.VMEM((2, N_PAD), jnp.float32),
            pltpu.VMEM((2, N_PAD), jnp.float32),
            pltpu.VMEM((2, CH), jnp.int32),
            pltpu.VMEM((2, CH), jnp.int32),
            pltpu.SemaphoreType.DMA,
            pltpu.SemaphoreType.DMA,
        ],
    )


def _lx(vT, pc, y=None, sub=None):
    nh = vT.shape[0]
    v32 = vT.reshape(nh, 32, 2, N_PAD)
    args = [v32, pc]
    if y is not None:
        args.append(y.reshape(nh, 32, 2, N_PAD))
    if sub is not None:
        args.append(sub.reshape(nh, 32, 2, N_PAD))
    res = _lxt_call(nh, y is not None, sub is not None)(*args)
    return res.reshape(nh, 64, N_PAD)


def kernel(x, edge_index, edge_weight, adj_w1, adj_w2, conv1_w, conv1_b, conv2_w, conv2_b):
    ew = _edge_mlp(edge_weight, adj_w1, adj_w2)
    reps = edge_index.shape[-1] // 420
    train_ew = jnp.tile(ew, (reps, 1))

    pad = E_PAD - E
    rowp = jnp.concatenate([edge_index[0], jnp.zeros((pad,), edge_index.dtype)])
    colp = jnp.concatenate([edge_index[1], jnp.zeros((pad,), edge_index.dtype)])
    cp = jnp.concatenate([train_ew.reshape(-1), jnp.zeros((pad,), jnp.float32)])

    deg_parts = _deg_call()(rowp, cp).reshape(32, 200, 128)
    dis = _dis(deg_parts).reshape(N_PAD)
    pc1, pc2 = _norm_call()(dis, rowp, colp, cp)

    t0 = _xpose(x)
    t1 = _lx(t0, pc1)
    t2 = _lx(t1, pc2, sub=t0)
    t3 = _lx(t2, pc2, sub=t1)
    t4 = _lx(t3, pc2, sub=t2)
    h = _conv1_mm([t0, t1, t2, t3, t4], conv1_w, conv1_b)

    yy = _proj_mm(h, conv2_w)
    b4 = yy[4:5]
    b3 = _lx(b4, pc2, y=yy[3:4])
    b2 = _lx(b3, pc2, y=yy[2:3], sub=b4)
    b1 = _lx(b2, pc2, y=yy[1:2], sub=b3)
    outT = _lx(b1, pc1, y=yy[0:1], sub=b2)
    out = _unpose(outT, conv2_b)

    return (out, ew, train_ew)


# speedup vs baseline: 1.0822x; 1.0483x over previous
import functools

import jax
import jax.numpy as jnp
from jax import lax
from jax.experimental import pallas as pl
from jax.experimental.pallas import tpu as pltpu
from jax.experimental.pallas import tpu_sc as plsc

N = 25200
E = 504000
N_PAD = 25600
E_PAD = 524288
EPT = E_PAD // 32
NCH = EPT // 128
CH = 4096
NPAIR = E_PAD // CH // 2
NB = 1280
GN = N_PAD // NB

_SC_PARAMS = pltpu.CompilerParams(needs_layout_passes=False,
                                  use_tc_tiling_on_sc=False)


@functools.cache
def _mesh():
    return plsc.VectorSubcoreMesh(
        core_axis_name="c", subcore_axis_name="s", num_cores=2, num_subcores=16)


def _wid():
    return lax.axis_index("c") * 16 + lax.axis_index("s")


def _mlp_kernel(ew_ref, w1_ref, w2_ref, out_ref):
    t = ew_ref[...].reshape(1, -1)
    t = t @ w1_ref[...].T
    t = jnp.where(t > 0, t, jnp.exp(t) - 1.0)
    t = t @ w2_ref[...].T
    t = jnp.tanh(t)
    t = jnp.maximum(t, 0.0)
    out_ref[...] = t.reshape(-1, 1)


def _edge_mlp(edge_weight, adj_w1, adj_w2):
    return pl.pallas_call(
        _mlp_kernel,
        out_shape=jax.ShapeDtypeStruct((420, 1), jnp.float32),
    )(edge_weight, adj_w1, adj_w2)


def _dis_kernel(degs_ref, out_ref):
    deg = jnp.sum(degs_ref[...], axis=0)
    out_ref[...] = jnp.where(deg > 0, lax.rsqrt(deg), 0.0)


def _dis(deg_parts):
    return pl.pallas_call(
        _dis_kernel,
        out_shape=jax.ShapeDtypeStruct((200, 128), jnp.float32),
    )(deg_parts)


def _xpose_kernel(x_ref, out_ref):
    xt = x_ref[...].T
    z = jnp.zeros((64, N_PAD - N), jnp.float32)
    out_ref[0] = jnp.concatenate([xt[:64], z], axis=1)
    out_ref[1] = jnp.concatenate([xt[64:], z], axis=1)


def _xpose(x):
    return pl.pallas_call(
        _xpose_kernel,
        out_shape=jax.ShapeDtypeStruct((2, 64, N_PAD), jnp.float32),
    )(x)


def _unpose_kernel(a_ref, b_ref, out_ref):
    out_ref[...] = a_ref[0][:, :N].T + b_ref[...].reshape(1, 64)


def _unpose(a, bias):
    return pl.pallas_call(
        _unpose_kernel,
        out_shape=jax.ShapeDtypeStruct((N, 64), jnp.float32),
    )(a, bias.reshape(64, 1))


def _comb_kernel(*refs, has_y, has_sub):
    i = 0
    a = refs[i][...]; i += 1
    if has_y:
        a = a + refs[i][...]; i += 1
    if has_sub:
        a = a - refs[i][...]; i += 1
    refs[i][...] = a


def _comb(a, y=None, ycol=None, sub=None):
    has_y, has_sub = y is not None, sub is not None
    H = a.shape[0]
    in_specs = [pl.BlockSpec((H, 64, NB), lambda i: (0, 0, i))]
    args = [a]
    if has_y:
        in_specs.append(pl.BlockSpec((1, 64, NB), lambda i, c=ycol: (c, 0, i)))
        args.append(y)
    if has_sub:
        in_specs.append(pl.BlockSpec((H, 64, NB), lambda i: (0, 0, i)))
        args.append(sub)
    return pl.pallas_call(
        functools.partial(_comb_kernel, has_y=has_y, has_sub=has_sub),
        grid=(GN,),
        in_specs=in_specs,
        out_specs=pl.BlockSpec((H, 64, NB), lambda i: (0, 0, i)),
        out_shape=jax.ShapeDtypeStruct((H, 64, N_PAD), jnp.float32),
    )(*args)


def _conv1_mm_kernel(*refs):
    ts = refs[:5]
    w, b, out_ref = refs[5], refs[6], refs[7]
    wv = w[...]
    acc = None
    for k in range(5):
        tk = ts[k][...]
        for h in range(2):
            wk = wv[k, 64 * h:64 * h + 64, :]
            d = lax.dot_general(wk, tk[h],
                                (((0,), (0,)), ((), ())),
                                preferred_element_type=jnp.float32)
            acc = d if acc is None else acc + d
    out_ref[...] = jnp.maximum(acc + b[...], 0.0)


def _conv1_mm(ts, w, b):
    t_spec = pl.BlockSpec((2, 64, NB), lambda i: (0, 0, i))
    return pl.pallas_call(
        _conv1_mm_kernel,
        grid=(GN,),
        in_specs=[t_spec] * 5 + [
            pl.BlockSpec((5, 128, 256), lambda i: (0, 0, 0)),
            pl.BlockSpec((256, 1), lambda i: (0, 0)),
        ],
        out_specs=pl.BlockSpec((256, NB), lambda i: (0, i)),
        out_shape=jax.ShapeDtypeStruct((256, N_PAD), jnp.float32),
    )(*ts, w, b.reshape(256, 1))


def _proj_mm_kernel(h, w, out_ref):
    out_ref[0] = lax.dot_general(w[0], h[...], (((0,), (0,)), ((), ())),
                                 preferred_element_type=jnp.float32)


def _proj_mm(h, w):
    return pl.pallas_call(
        _proj_mm_kernel,
        grid=(5, GN),
        in_specs=[
            pl.BlockSpec((256, NB), lambda k, i: (0, i)),
            pl.BlockSpec((1, 256, 64), lambda k, i: (k, 0, 0)),
        ],
        out_specs=pl.BlockSpec((1, 64, NB), lambda k, i: (k, 0, i)),
        out_shape=jax.ShapeDtypeStruct((5, 64, N_PAD), jnp.float32),
    )(h, w)


def _deg_body(rowp, coefp, out, dpriv, idxb, cb):
    w = _wid()
    def zero(i, _):
        dpriv[pl.ds(16 * i, 16)] = jnp.zeros((16,), jnp.float32)
        return 0
    lax.fori_loop(0, N_PAD // 16, zero, 0)
    base = w * EPT
    def chunk(k, _):
        e0 = base + k * 128
        pltpu.sync_copy(rowp.at[pl.ds(e0, 128)], idxb)
        pltpu.sync_copy(coefp.at[pl.ds(e0, 128)], cb)
        @plsc.parallel_loop(0, 128, 16, unroll=4)
        def _(g):
            r16 = idxb[pl.ds(g, 16)]
            c16 = cb[pl.ds(g, 16)]
            plsc.addupdate_scatter(dpriv, [r16], c16)
        return 0
    lax.fori_loop(0, NCH, chunk, 0)
    pltpu.sync_copy(dpriv, out.at[pl.ds(w * N_PAD, N_PAD)])


@functools.cache
def _deg_call():
    return pl.kernel(
        _deg_body,
        out_type=jax.ShapeDtypeStruct((32 * N_PAD,), jnp.float32),
        mesh=_mesh(),
        compiler_params=_SC_PARAMS,
        scratch_types=[
            pltpu.VMEM((N_PAD,), jnp.float32),
            pltpu.VMEM((128,), jnp.int32),
            pltpu.VMEM((128,), jnp.float32),
        ],
    )


def _norm_body(dis, rowp, colp, coefp, pc1_out, pc2_out,
               disv, idxr, idxc, cb, pkb, na, nb):
    w = _wid()
    pltpu.sync_copy(dis, disv)
    base = w * EPT
    def chunk(k, _):
        e0 = base + k * 128
        pltpu.sync_copy(rowp.at[pl.ds(e0, 128)], idxr)
        pltpu.sync_copy(colp.at[pl.ds(e0, 128)], idxc)
        pltpu.sync_copy(coefp.at[pl.ds(e0, 128)], cb)
        @plsc.parallel_loop(0, 128, 16, unroll=4)
        def _(g):
            sl = pl.ds(g, 16)
            r16 = idxr[sl]
            c16 = idxc[sl]
            w16 = cb[sl]
            dr = plsc.load_gather(disv, [r16])
            dc = plsc.load_gather(disv, [c16])
            v = -(dr * w16 * dc)
            pkb[sl] = jnp.bitwise_or(r16, jnp.left_shift(c16, 16))
            na[sl] = plsc.bitcast(v, jnp.int32)
            nb[sl] = plsc.bitcast(v + v, jnp.int32)
        pltpu.sync_copy(pkb, pc1_out.at[0, pl.ds(e0, 128)])
        pltpu.sync_copy(pkb, pc2_out.at[0, pl.ds(e0, 128)])
        pltpu.sync_copy(na, pc1_out.at[1, pl.ds(e0, 128)])
        pltpu.sync_copy(nb, pc2_out.at[1, pl.ds(e0, 128)])
        return 0
    lax.fori_loop(0, NCH, chunk, 0)


@functools.cache
def _norm_call():
    return pl.kernel(
        _norm_body,
        out_type=(jax.ShapeDtypeStruct((2, E_PAD), jnp.int32),
                  jax.ShapeDtypeStruct((2, E_PAD), jnp.int32)),
        mesh=_mesh(),
        compiler_params=_SC_PARAMS,
        scratch_types=[
            pltpu.VMEM((N_PAD,), jnp.float32),
            pltpu.VMEM((128,), jnp.int32),
            pltpu.VMEM((128,), jnp.int32),
            pltpu.VMEM((128,), jnp.float32),
            pltpu.VMEM((128,), jnp.int32),
            pltpu.VMEM((128,), jnp.int32),
            pltpu.VMEM((128,), jnp.int32),
        ],
    )


def _lxt_body(*refs, nh, has_y, has_sub):
    i = 0
    vT = refs[i]; i += 1
    pcp = refs[i]; i += 1
    y = sub = None
    if has_y:
        y = refs[i]; i += 1
    if has_sub:
        sub = refs[i]; i += 1
    out, vbuf, abuf, pc0, pc1, s0, s1 = refs[i:i + 7]
    w = _wid()
    f0 = jnp.zeros((16,), jnp.int32)
    f1 = jnp.full((16,), 1, jnp.int32)

    def issue(k, pcb, sem):
        pltpu.async_copy(pcp.at[:, pl.ds(k * CH, CH)], pcb, sem)

    def process(k, pcb, sem, more):
        pltpu.make_async_copy(pcp.at[:, pl.ds(k * CH, CH)], pcb, sem).wait()
        @plsc.parallel_loop(0, CH, 64, unroll=8)
        def _(i):
            for u in range(4):
                sl = pl.ds(i + 16 * u, 16)
                pk16 = pcb[0, sl]
                c16 = plsc.bitcast(pcb[1, sl], jnp.float32)
                r16 = jnp.bitwise_and(pk16, 0xFFFF)
                o16 = lax.shift_right_logical(pk16, 16)
                v0 = plsc.load_gather(vbuf, [f0, r16])
                plsc.addupdate_scatter(abuf, [f0, o16], v0 * c16)
                v1 = plsc.load_gather(vbuf, [f1, r16])
                plsc.addupdate_scatter(abuf, [f1, o16], v1 * c16)
        @pl.when(more)
        def _():
            issue(k + 2, pcb, sem)

    for h in range(nh):
        pltpu.sync_copy(vT.at[h, w], vbuf)
        def zero(i, _):
            sl = pl.ds(16 * i, 16)
            z = jnp.zeros((16,), jnp.float32)
            abuf[0, sl] = z
            abuf[1, sl] = z
            return 0
        lax.fori_loop(0, N_PAD // 16, zero, 0)
        issue(0, pc0, s0)
        issue(1, pc1, s1)
        def pair(m, _):
            more = m < NPAIR - 1
            process(2 * m, pc0, s0, more)
            process(2 * m + 1, pc1, s1, more)
            return 0
        lax.fori_loop(0, NPAIR, pair, 0)
        for arr, sign in ((y, 1.0), (sub, -1.0)):
            if arr is not None:
                pltpu.sync_copy(arr.at[h, w], vbuf)
                @plsc.parallel_loop(0, N_PAD, 16, unroll=4)
                def _(i):
                    sl = pl.ds(i, 16)
                    if sign > 0:
                        abuf[0, sl] = abuf[0, sl] + vbuf[0, sl]
                        abuf[1, sl] = abuf[1, sl] + vbuf[1, sl]
                    else:
                        abuf[0, sl] = abuf[0, sl] - vbuf[0, sl]
                        abuf[1, sl] = abuf[1, sl] - vbuf[1, sl]
        pltpu.sync_copy(abuf, out.at[h, w])


@functools.cache
def _lxt_call(nh, has_y, has_sub):
    return pl.kernel(
        functools.partial(_lxt_body, nh=nh, has_y=has_y, has_sub=has_sub),
        out_type=jax.ShapeDtypeStruct((nh, 32, 2, N_PAD), jnp.float32),
        mesh=_mesh(),
        compiler_params=_SC_PARAMS,
        scratch_types=[
            pltpu.VMEM((2, N_PAD), jnp.float32),
            pltpu.VMEM((2, N_PAD), jnp.float32),
            pltpu.VMEM((2, CH), jnp.int32),
            pltpu.VMEM((2, CH), jnp.int32),
            pltpu.SemaphoreType.DMA,
            pltpu.SemaphoreType.DMA,
        ],
    )


def _lx(vT, pc, y=None, sub=None):
    nh = vT.shape[0]
    v32 = vT.reshape(nh, 32, 2, N_PAD)
    args = [v32, pc]
    if y is not None:
        args.append(y.reshape(nh, 32, 2, N_PAD))
    if sub is not None:
        args.append(sub.reshape(nh, 32, 2, N_PAD))
    res = _lxt_call(nh, y is not None, sub is not None)(*args)
    return res.reshape(nh, 64, N_PAD)


def kernel(x, edge_index, edge_weight, adj_w1, adj_w2, conv1_w, conv1_b, conv2_w, conv2_b):
    ew = _edge_mlp(edge_weight, adj_w1, adj_w2)
    reps = edge_index.shape[-1] // 420
    train_ew = jnp.tile(ew, (reps, 1))

    pad = E_PAD - E
    rowp = jnp.concatenate([edge_index[0], jnp.zeros((pad,), edge_index.dtype)])
    colp = jnp.concatenate([edge_index[1], jnp.zeros((pad,), edge_index.dtype)])
    cp = jnp.concatenate([train_ew.reshape(-1), jnp.zeros((pad,), jnp.float32)])

    deg_parts = _deg_call()(rowp, cp).reshape(32, 200, 128)
    dis = _dis(deg_parts).reshape(N_PAD)
    pc1, pc2 = _norm_call()(dis, rowp, colp, cp)

    t0 = _xpose(x)
    t1 = _lx(t0, pc1)
    t2 = _lx(t1, pc2, sub=t0)
    t3 = _lx(t2, pc2, sub=t1)
    t4 = _lx(t3, pc2, sub=t2)
    h = _conv1_mm([t0, t1, t2, t3, t4], conv1_w, conv1_b)

    yy = _proj_mm(h, conv2_w)
    b4 = yy[4:5]
    b3 = _lx(b4, pc2, y=yy[3:4])
    b2 = _lx(b3, pc2, y=yy[2:3], sub=b4)
    b1 = _lx(b2, pc2, y=yy[1:2], sub=b3)
    outT = _lx(b1, pc1, y=yy[0:1], sub=b2)
    out = _unpose(outT, conv2_b)

    return (out, ew, train_ew)
